# Initial kernel scaffold; baseline (speedup 1.0000x reference)
import jax
import jax.numpy as jnp
from jax.experimental import pallas as pl


def kernel(inputs, loc_table, scale_table):
    raise NotImplementedError("write your pallas kernel here")


# speedup vs baseline: 1.2911x; 1.2911x over previous
import functools

import jax
import jax.numpy as jnp
from jax import lax
from jax.experimental import pallas as pl
from jax.experimental.pallas import tpu as pltpu
from jax.experimental.pallas import tpu_sc as plsc

EMBED_DIM = 32
BATCH = 16384
HIST = 50
B_TOTAL = BATCH * HIST
NUM_WORKERS = 32
B_PER_W = B_TOTAL // NUM_WORKERS
CHUNK = 128
NCHUNK = B_PER_W // CHUNK

_mesh = plsc.VectorSubcoreMesh(core_axis_name="c", subcore_axis_name="s")


@functools.partial(
    pl.kernel,
    mesh=_mesh,
    compiler_params=pltpu.CompilerParams(use_tc_tiling_on_sc=False),
    out_type=[
        jax.ShapeDtypeStruct((NUM_WORKERS, NCHUNK, CHUNK, EMBED_DIM), jnp.float32),
        jax.ShapeDtypeStruct((NUM_WORKERS, NCHUNK, CHUNK, EMBED_DIM), jnp.float32),
    ],
    scratch_types=[
        pltpu.VMEM((NCHUNK, CHUNK), jnp.int32),
        pltpu.VMEM((CHUNK, EMBED_DIM), jnp.float32),
        pltpu.VMEM((CHUNK, EMBED_DIM), jnp.float32),
        pltpu.SemaphoreType.DMA,
    ],
)
def _gather_kernel(idx_hbm, loc_hbm, scale_hbm, out_loc, out_scale,
                   idx_v, loc_rows, scale_rows, sem):
    wid = lax.axis_index("s") * 2 + lax.axis_index("c")
    pltpu.sync_copy(idx_hbm.at[wid], idx_v)

    def body(j, carry):
        cp_loc = pltpu.async_copy(loc_hbm.at[idx_v.at[j]], loc_rows, sem)
        cp_scale = pltpu.async_copy(scale_hbm.at[idx_v.at[j]], scale_rows, sem)
        cp_loc.wait()
        cp_scale.wait()
        pltpu.sync_copy(loc_rows, out_loc.at[wid, j])
        pltpu.sync_copy(scale_rows, out_scale.at[wid, j])
        return carry

    lax.fori_loop(0, NCHUNK, body, 0)


def kernel(inputs, loc_table, scale_table):
    idx = inputs.astype(jnp.int32).reshape(NUM_WORKERS, NCHUNK, CHUNK)
    out_loc, out_scale = _gather_kernel(idx, loc_table, scale_table)
    return (out_loc.reshape(BATCH, HIST, EMBED_DIM),
            out_scale.reshape(BATCH, HIST, EMBED_DIM))


# speedup vs baseline: 1.3732x; 1.0636x over previous
import functools

import jax
import jax.numpy as jnp
from jax import lax
from jax.experimental import pallas as pl
from jax.experimental.pallas import tpu as pltpu
from jax.experimental.pallas import tpu_sc as plsc

EMBED_DIM = 32
BATCH = 16384
HIST = 50
B_TOTAL = BATCH * HIST
NUM_WORKERS = 32
B_PER_W = B_TOTAL // NUM_WORKERS
CHUNK = 128
NCHUNK = B_PER_W // CHUNK

NBUF = 8

_mesh = plsc.VectorSubcoreMesh(core_axis_name="c", subcore_axis_name="s")


@functools.partial(
    pl.kernel,
    mesh=_mesh,
    compiler_params=pltpu.CompilerParams(use_tc_tiling_on_sc=False),
    out_type=[
        jax.ShapeDtypeStruct((NUM_WORKERS, NCHUNK, CHUNK, EMBED_DIM), jnp.float32),
        jax.ShapeDtypeStruct((NUM_WORKERS, NCHUNK, CHUNK, EMBED_DIM), jnp.float32),
    ],
    scratch_types=[
        pltpu.VMEM((NCHUNK, CHUNK), jnp.int32),
        pltpu.VMEM((NBUF, CHUNK, EMBED_DIM), jnp.float32),
        pltpu.VMEM((NBUF, CHUNK, EMBED_DIM), jnp.float32),
        pltpu.SemaphoreType.DMA((NBUF,)),
    ],
)
def _gather_kernel(idx_hbm, loc_hbm, scale_hbm, out_loc, out_scale,
                   idx_v, loc_rows, scale_rows, gsem):
    wid = lax.axis_index("s") * 2 + lax.axis_index("c")
    pltpu.sync_copy(idx_hbm.at[wid], idx_v)

    def fire(j, b):
        pltpu.async_copy(loc_hbm.at[idx_v.at[j]], loc_rows.at[b], gsem.at[b])
        pltpu.async_copy(scale_hbm.at[idx_v.at[j]], scale_rows.at[b], gsem.at[b])

    for b in range(NBUF):
        fire(b, b)

    @pl.loop(0, NCHUNK, step=NBUF)
    def group(g):
        for b in range(NBUF):
            j = g + b
            pltpu.make_async_copy(
                loc_hbm.at[idx_v.at[j]], loc_rows.at[b], gsem.at[b]).wait()
            pltpu.make_async_copy(
                scale_hbm.at[idx_v.at[j]], scale_rows.at[b], gsem.at[b]).wait()
            pltpu.sync_copy(loc_rows.at[b], out_loc.at[wid, j])
            pltpu.sync_copy(scale_rows.at[b], out_scale.at[wid, j])

            @pl.when(j + NBUF < NCHUNK)
            def refire():
                fire(j + NBUF, b)


def kernel(inputs, loc_table, scale_table):
    idx = inputs.astype(jnp.int32).reshape(NUM_WORKERS, NCHUNK, CHUNK)
    out_loc, out_scale = _gather_kernel(idx, loc_table, scale_table)
    return (out_loc.reshape(BATCH, HIST, EMBED_DIM),
            out_scale.reshape(BATCH, HIST, EMBED_DIM))
